# baseline (speedup 1.0000x reference)
import functools

import jax
import jax.numpy as jnp
from jax.experimental import pallas as pl
from jax.experimental.pallas import tpu as pltpu


def _rollout_kernel(x_ref, h0_ref, w1_ref, b1_ref, wih_ref, bih_ref,
                    whh_ref, bhh_ref, bsum_ref, w2_ref, b2_ref,
                    q_ref, hout_ref, h_scr, gi_scr, *, chunk):
    t = pl.program_id(0)
    H = h_scr.shape[-1]

    @pl.when(t == 0)
    def _():
        h_scr[...] = h0_ref[...]

    w1 = w1_ref[...]
    wih = wih_ref[...]
    b1 = b1_ref[...]

    for i in range(chunk):
        x = x_ref[i]
        a = jnp.dot(x, w1, preferred_element_type=jnp.float32) + b1
        a = jnp.maximum(a, 0.0).astype(jnp.bfloat16)
        gi_scr[i] = jnp.dot(a, wih, preferred_element_type=jnp.float32)

    whh = whh_ref[...]
    w2 = w2_ref[...]
    bsum = bsum_ref[...]
    bih_n = bih_ref[:, 2 * H:3 * H]
    bhh_n = bhh_ref[:, 2 * H:3 * H]
    b2 = b2_ref[...]
    h = h_scr[...]

    for i in range(chunk):
        gh = jnp.dot(h.astype(jnp.bfloat16), whh,
                     preferred_element_type=jnp.float32)
        g = gi_scr[i] + gh + bsum
        r = jax.nn.sigmoid(g[:, 0:H])
        z = jax.nn.sigmoid(g[:, H:2 * H])
        n = jnp.tanh(gi_scr[i][:, 2 * H:3 * H] + bih_n
                     + r * (gh[:, 2 * H:3 * H] + bhh_n))
        h = (1.0 - z) * n + z * h
        q_ref[i] = jnp.dot(h.astype(jnp.bfloat16), w2,
                           preferred_element_type=jnp.float32) + b2

    h_scr[...] = h
    hout_ref[...] = h


def _rollout(x_seq, h0, w1t, b1, w_ih_t, b_ih, w_hh_t, b_hh, w2t, b2,
             *, chunk):
    T, B, in_dim = x_seq.shape
    H = h0.shape[1]
    A = w2t.shape[1]
    assert T % chunk == 0

    bf = jnp.bfloat16
    x_seq = x_seq.astype(bf)
    w1b, wihb, whhb, w2b = (w.astype(bf) for w in (w1t, w_ih_t, w_hh_t, w2t))
    bsum = b_ih + b_hh

    def wspec(arr):
        return pl.BlockSpec(arr.shape, lambda t: (0,) * arr.ndim)

    grid = (T // chunk,)
    body = functools.partial(_rollout_kernel, chunk=chunk)

    q_seq, h_final = pl.pallas_call(
        body,
        out_shape=(
            jax.ShapeDtypeStruct((T, B, A), jnp.float32),
            jax.ShapeDtypeStruct((B, H), jnp.float32),
        ),
        grid_spec=pltpu.PrefetchScalarGridSpec(
            num_scalar_prefetch=0,
            grid=grid,
            in_specs=[
                pl.BlockSpec((chunk, B, in_dim), lambda t: (t, 0, 0)),
                pl.BlockSpec((B, H), lambda t: (0, 0)),
                wspec(w1b), wspec(b1),
                wspec(wihb), wspec(b_ih),
                wspec(whhb), wspec(b_hh), wspec(bsum),
                wspec(w2b), wspec(b2),
            ],
            out_specs=(
                pl.BlockSpec((chunk, B, A), lambda t: (t, 0, 0)),
                pl.BlockSpec((B, H), lambda t: (0, 0)),
            ),
            scratch_shapes=[
                pltpu.VMEM((B, H), jnp.float32),
                pltpu.VMEM((chunk, B, 3 * H), jnp.float32),
            ],
        ),
        compiler_params=pltpu.CompilerParams(
            dimension_semantics=("arbitrary",)),
    )(
        x_seq, h0,
        w1b, b1, wihb, b_ih, whhb, b_hh, bsum, w2b, b2,
    )
    return q_seq, h_final


def kernel(x_seq, h0, w1t, b1, w_ih_t, b_ih, w_hh_t, b_hh, w2t, b2):
    return _rollout(x_seq, h0, w1t, b1, w_ih_t, b_ih, w_hh_t, b_hh, w2t, b2,
                    chunk=8)


# speedup vs baseline: 1.2182x; 1.2182x over previous
import functools

import jax
import jax.numpy as jnp
from jax.experimental import pallas as pl
from jax.experimental.pallas import tpu as pltpu


def _rollout_kernel(x_ref, h0_ref, w1_ref, b1_ref, wih_ref, bih_ref,
                    wcat_ref, bhh_ref, bsum_ref, w2_ref, b2_ref,
                    q_ref, hout_ref, h_scr, gi_scr, *, chunk):
    t = pl.program_id(0)
    H = h_scr.shape[-1]

    @pl.when(t == 0)
    def _():
        h_scr[...] = h0_ref[...]

    w1 = w1_ref[...]
    wih = wih_ref[...]
    b1 = b1_ref[...]

    for i in range(chunk):
        x = x_ref[i]
        a = jnp.dot(x, w1, preferred_element_type=jnp.float32) + b1
        a = jnp.maximum(a, 0.0).astype(jnp.bfloat16)
        gi_scr[i] = jnp.dot(a, wih, preferred_element_type=jnp.float32)

    wcat = wcat_ref[...]
    w2 = w2_ref[...]
    bsum = bsum_ref[...]
    bih_n = bih_ref[:, 2 * H:3 * H]
    bhh_n = bhh_ref[:, 2 * H:3 * H]
    b2 = b2_ref[...]
    A = q_ref.shape[-1]
    h = h_scr[...]

    for i in range(chunk):
        hb = h.astype(jnp.bfloat16)
        c = jnp.dot(hb, wcat, preferred_element_type=jnp.float32)
        if i > 0:
            q_ref[i - 1] = c[:, 3 * H:3 * H + A] + b2
        r = jax.nn.sigmoid(gi_scr[i, :, 0:H] + c[:, 0:H] + bsum[:, 0:H])
        z = jax.nn.sigmoid(gi_scr[i, :, H:2 * H] + c[:, H:2 * H]
                           + bsum[:, H:2 * H])
        n = jnp.tanh(gi_scr[i, :, 2 * H:3 * H] + bih_n
                     + r * (c[:, 2 * H:3 * H] + bhh_n))
        h = (1.0 - z) * n + z * h

    q_ref[chunk - 1] = jnp.dot(h.astype(jnp.bfloat16), w2,
                               preferred_element_type=jnp.float32) + b2
    h_scr[...] = h
    hout_ref[...] = h


def _rollout(x_seq, h0, w1t, b1, w_ih_t, b_ih, w_hh_t, b_hh, w2t, b2,
             *, chunk):
    T, B, in_dim = x_seq.shape
    H = h0.shape[1]
    A = w2t.shape[1]
    assert T % chunk == 0

    bf = jnp.bfloat16
    w1b, wihb, whhb, w2b = (w.astype(bf) for w in (w1t, w_ih_t, w_hh_t, w2t))
    w2pad = jnp.zeros((H, 128), bf).at[:, :A].set(w2b)
    wcat = jnp.concatenate([whhb, w2pad], axis=1)
    bsum = b_ih + b_hh

    def wspec(arr):
        return pl.BlockSpec(arr.shape, lambda t: (0,) * arr.ndim)

    grid = (T // chunk,)
    body = functools.partial(_rollout_kernel, chunk=chunk)

    q_seq, h_final = pl.pallas_call(
        body,
        out_shape=(
            jax.ShapeDtypeStruct((T, B, A), jnp.float32),
            jax.ShapeDtypeStruct((B, H), jnp.float32),
        ),
        grid_spec=pltpu.PrefetchScalarGridSpec(
            num_scalar_prefetch=0,
            grid=grid,
            in_specs=[
                pl.BlockSpec((chunk, B, in_dim), lambda t: (t, 0, 0)),
                pl.BlockSpec((B, H), lambda t: (0, 0)),
                wspec(w1b), wspec(b1),
                wspec(wihb), wspec(b_ih),
                wspec(wcat), wspec(b_hh), wspec(bsum),
                wspec(w2b), wspec(b2),
            ],
            out_specs=(
                pl.BlockSpec((chunk, B, A), lambda t: (t, 0, 0)),
                pl.BlockSpec((B, H), lambda t: (0, 0)),
            ),
            scratch_shapes=[
                pltpu.VMEM((B, H), jnp.float32),
                pltpu.VMEM((chunk, B, 3 * H), jnp.float32),
            ],
        ),
        compiler_params=pltpu.CompilerParams(
            dimension_semantics=("arbitrary",)),
    )(
        x_seq, h0,
        w1b, b1, wihb, b_ih, wcat, b_hh, bsum, w2b, b2,
    )
    return q_seq, h_final


def kernel(x_seq, h0, w1t, b1, w_ih_t, b_ih, w_hh_t, b_hh, w2t, b2):
    return _rollout(x_seq, h0, w1t, b1, w_ih_t, b_ih, w_hh_t, b_hh, w2t, b2,
                    chunk=8)
